# Initial kernel scaffold; baseline (speedup 1.0000x reference)
import jax
import jax.numpy as jnp
from jax.experimental import pallas as pl


def kernel(pha, amp):
    raise NotImplementedError("write your pallas kernel here")


# speedup vs baseline: 47.4643x; 47.4643x over previous
import functools

import numpy as np
import jax
import jax.numpy as jnp
from jax import lax
from jax.experimental import pallas as pl
from jax.experimental.pallas import tpu as pltpu
from jax.experimental.pallas import tpu_sc as plsc

NB = 18
FP = 8
FA = 8
T = 512
NW = 32
L = 16
CHUNKS = T // L
CNTS_W = FP * NB * L
SUMS_W = FA * FP * NB * L

def _sc_hist_body(pha_hbm, amp_hbm, cut_hbm, sums_hbm, cnts_hbm,
                  pha_v, amp_v, cut_v, sums_v, cnts_v):
    wid = lax.axis_index("s") * 2 + lax.axis_index("c")
    pltpu.sync_copy(pha_hbm.at[wid], pha_v)
    pltpu.sync_copy(amp_hbm.at[wid], amp_v)
    pltpu.sync_copy(cut_hbm, cut_v)

    zf = jnp.zeros((L,), jnp.float32)

    def zero_sums(i, c):
        base = i * (16 * L)
        for q in range(16):
            sums_v[pl.ds(base + q * L, L)] = zf
        return c

    lax.fori_loop(0, SUMS_W // (16 * L), zero_sums, 0)

    def zero_cnts(i, c):
        base = i * (16 * L)
        for q in range(16):
            cnts_v[pl.ds(base + q * L, L)] = zf
        return c

    lax.fori_loop(0, CNTS_W // (16 * L), zero_cnts, 0)

    lane = lax.iota(jnp.int32, L)
    ones_f = jnp.ones((L,), jnp.float32)
    ones_i = jnp.ones((L,), jnp.int32)
    zero_i = jnp.zeros((L,), jnp.int32)
    pi = jnp.float32(np.pi)
    inv = jnp.float32(NB / (2.0 * np.pi))

    def chunk(ci, c):
        t0 = ci * L
        avecs = [amp_v[pl.ds(fa * T + t0, L)] for fa in range(FA)]
        for fp in range(FP):
            p = pha_v[pl.ds(fp * T + t0, L)]
            y = (p + pi) * inv
            y = jnp.minimum(jnp.maximum(y, jnp.float32(-1.0)), jnp.float32(19.0))
            k = (y + jnp.float32(32.0)).astype(jnp.int32) - 32
            j = jnp.minimum(jnp.maximum(k, 0), NB - 1)
            cj = plsc.load_gather(cut_v, [j])
            cj1 = plsc.load_gather(cut_v, [j + 1])
            u = (j + jnp.where(cj < p, ones_i, zero_i)
                 + jnp.where(cj1 < p, ones_i, zero_i))
            b = jnp.minimum(jnp.maximum(u - 1, 0), NB - 1)
            cbase = b * L + lane
            plsc.addupdate_scatter(cnts_v, [fp * (NB * L) + cbase], ones_f)
            for fa in range(FA):
                plsc.addupdate_scatter(
                    sums_v, [(fa * FP + fp) * (NB * L) + cbase], avecs[fa])
        return c

    lax.fori_loop(0, CHUNKS, chunk, 0)

    pltpu.sync_copy(sums_v, sums_hbm.at[wid])
    pltpu.sync_copy(cnts_v, cnts_hbm.at[wid])


@functools.cache
def _sc_hist():
    mesh = plsc.VectorSubcoreMesh(core_axis_name="c", subcore_axis_name="s")
    return pl.kernel(
        _sc_hist_body,
        out_type=(jax.ShapeDtypeStruct((NW, SUMS_W), jnp.float32),
                  jax.ShapeDtypeStruct((NW, CNTS_W), jnp.float32)),
        mesh=mesh,
        compiler_params=pltpu.CompilerParams(needs_layout_passes=False),
        scratch_types=(pltpu.VMEM((FP * T,), jnp.float32),
                       pltpu.VMEM((FA * T,), jnp.float32),
                       pltpu.VMEM((32,), jnp.float32),
                       pltpu.VMEM((SUMS_W,), jnp.float32),
                       pltpu.VMEM((CNTS_W,), jnp.float32)),
    )


_cols = np.arange(2 * NB * L)
_M1 = np.zeros((2 * NB * L, 2 * NB), np.float32)
_M1[_cols, (_cols // (NB * L)) * NB + (_cols // L) % NB] = 1.0
_K1 = np.zeros((2 * NB, 2), np.float32)
_K1[np.arange(2 * NB), np.arange(2 * NB) // NB] = 1.0
_K2 = np.ascontiguousarray(_K1.T)
_KH = np.full((2, 1), 0.5, np.float32)
_LOG_NUM = float(np.log(np.float32(NB) + np.float32(1e-9)))
_LOG_DEN = float(np.log(np.float32(NB)))


def _tc_body(xs_ref, xc_ref, m1_ref, k1_ref, k2_ref, kh_ref, out_ref):
    eps = jnp.float32(1e-9)
    s = jnp.dot(xs_ref[...], m1_ref[...], preferred_element_type=jnp.float32, precision=lax.Precision.HIGHEST)
    c = jnp.dot(xc_ref[...], m1_ref[...], preferred_element_type=jnp.float32, precision=lax.Precision.HIGHEST)
    cf = jnp.concatenate([c] * FA, axis=0)
    means = s / (cf + eps)
    r = jnp.dot(means, k1_ref[...], preferred_element_type=jnp.float32, precision=lax.Precision.HIGHEST)
    rb = jnp.dot(r, k2_ref[...], preferred_element_type=jnp.float32, precision=lax.Precision.HIGHEST)
    probs = means / (rb + eps)
    plogp = probs * jnp.log(probs + eps)
    e = jnp.dot(plogp, k1_ref[...], preferred_element_type=jnp.float32, precision=lax.Precision.HIGHEST)
    mi = (jnp.float32(_LOG_NUM) + e) / jnp.float32(_LOG_DEN)
    out_ref[...] = jnp.dot(mi, kh_ref[...], preferred_element_type=jnp.float32, precision=lax.Precision.HIGHEST)


def kernel(pha, amp):
    pha = pha.astype(jnp.float32)
    amp = amp.astype(jnp.float32)
    phat = pha.transpose(0, 1, 3, 2, 4).reshape(NW, FP * T)
    ampt = amp.transpose(0, 1, 3, 2, 4).reshape(NW, FA * T)
    cut = jnp.linspace(-np.pi, np.pi, NB + 1).astype(jnp.float32)
    cutp = jnp.concatenate([cut, jnp.full((32 - (NB + 1),), 1e30, jnp.float32)])

    sums, cnts = _sc_hist()(phat, ampt, cutp)

    xs = (sums.reshape(2, 8, 2, FA, FP, NB, L)
          .transpose(3, 0, 1, 4, 2, 5, 6).reshape(FA * 128, 2 * NB * L))
    xc = (cnts.reshape(2, 8, 2, FP, NB, L)
          .transpose(0, 1, 3, 2, 4, 5).reshape(128, 2 * NB * L))

    mi = pl.pallas_call(
        _tc_body,
        out_shape=jax.ShapeDtypeStruct((FA * 128, 1), jnp.float32),
    )(xs, xc, jnp.asarray(_M1), jnp.asarray(_K1), jnp.asarray(_K2),
      jnp.asarray(_KH))

    return mi.reshape(FA, 2, 8, FP).transpose(1, 2, 3, 0)


# speedup vs baseline: 77.4340x; 1.6314x over previous
import functools

import numpy as np
import jax
import jax.numpy as jnp
from jax import lax
from jax.experimental import pallas as pl
from jax.experimental.pallas import tpu as pltpu
from jax.experimental.pallas import tpu_sc as plsc

NB = 18
FP = 8
FA = 8
T = 512
NW = 32
L = 16
CHUNKS = T // L
BLK = NB * L
CNTS_W = FP * BLK
SUMS_W = FP * FA * BLK


def _sc_hist_body(pha_hbm, amp_hbm, cut_hbm, sums_hbm, cnts_hbm,
                  pha_v, amp_v, cut_v, sums_v, cnts_v, sem):
    wid = lax.axis_index("s") * 2 + lax.axis_index("c")
    bc = wid // 2
    seg = wid % 2

    copies = [pltpu.async_copy(cut_hbm, cut_v, sem)]
    for fp in range(FP):
        copies.append(pltpu.async_copy(
            pha_hbm.at[bc * FP + fp, seg], pha_v.at[pl.ds(fp * T, T)], sem))
    for fa in range(FA):
        copies.append(pltpu.async_copy(
            amp_hbm.at[bc * FA + fa, seg], amp_v.at[pl.ds(fa * T, T)], sem))

    zf = jnp.zeros((L,), jnp.float32)

    def zero_sums(i, c):
        base = i * (16 * L)
        for q in range(16):
            sums_v[pl.ds(base + q * L, L)] = zf
        return c

    lax.fori_loop(0, SUMS_W // (16 * L), zero_sums, 0)

    def zero_cnts(i, c):
        base = i * (16 * L)
        for q in range(16):
            cnts_v[pl.ds(base + q * L, L)] = zf
        return c

    lax.fori_loop(0, CNTS_W // (16 * L), zero_cnts, 0)

    for cp in copies:
        cp.wait()

    lane = lax.iota(jnp.int32, L)
    ones_f = jnp.ones((L,), jnp.float32)
    ones_i = jnp.ones((L,), jnp.int32)
    zero_i = jnp.zeros((L,), jnp.int32)
    pi = jnp.float32(np.pi)
    inv = jnp.float32(NB / (2.0 * np.pi))

    def chunk(ci, c):
        t0 = ci * L
        avecs = [amp_v[pl.ds(fa * T + t0, L)] for fa in range(FA)]
        for fp in range(FP):
            p = pha_v[pl.ds(fp * T + t0, L)]
            y = (p + pi) * inv
            y = jnp.minimum(jnp.maximum(y, jnp.float32(-1.0)), jnp.float32(19.0))
            k = (y + jnp.float32(32.0)).astype(jnp.int32) - 32
            j = jnp.minimum(jnp.maximum(k, 0), NB - 1)
            cj = plsc.load_gather(cut_v, [j])
            cj1 = plsc.load_gather(cut_v, [j + 1])
            u = (j + jnp.where(cj < p, ones_i, zero_i)
                 + jnp.where(cj1 < p, ones_i, zero_i))
            b = jnp.minimum(jnp.maximum(u - 1, 0), NB - 1)
            cbase = b * L + lane
            plsc.addupdate_scatter(cnts_v, [fp * BLK + cbase], ones_f)
            fbase = fp * (FA * BLK) + cbase
            for fa in range(FA):
                plsc.addupdate_scatter(sums_v, [fbase + fa * BLK], avecs[fa])
        return c

    lax.fori_loop(0, CHUNKS, chunk, 0)

    pltpu.sync_copy(sums_v, sums_hbm.at[wid])
    pltpu.sync_copy(cnts_v, cnts_hbm.at[wid])


@functools.cache
def _sc_hist():
    mesh = plsc.VectorSubcoreMesh(core_axis_name="c", subcore_axis_name="s")
    return pl.kernel(
        _sc_hist_body,
        out_type=(jax.ShapeDtypeStruct((NW, SUMS_W), jnp.float32),
                  jax.ShapeDtypeStruct((NW, CNTS_W), jnp.float32)),
        mesh=mesh,
        compiler_params=pltpu.CompilerParams(needs_layout_passes=False),
        scratch_types=(pltpu.VMEM((FP * T,), jnp.float32),
                       pltpu.VMEM((FA * T,), jnp.float32),
                       pltpu.VMEM((32,), jnp.float32),
                       pltpu.VMEM((SUMS_W,), jnp.float32),
                       pltpu.VMEM((CNTS_W,), jnp.float32),
                       pltpu.SemaphoreType.DMA),
    )


_M1 = np.zeros((BLK, NB), np.float32)
_M1[np.arange(BLK), np.arange(BLK) // L] = 1.0
_ONES18 = np.ones((NB, 1), np.float32)
_ROWS = np.arange(NW * FP * FA)
_R = np.zeros((NW * FP * FA, NW * FP), np.float32)
_R[_ROWS, _ROWS // FA] = 1.0
_P = np.zeros((NW * FP * FA // 2, NW * FP * FA), np.float32)
_O = np.arange(NW * FP * FA // 2)
for _s in (0, 1):
    _P[_O, (_O // 64) * 128 + _s * 64 + _O % 64] = 0.5
_LOG_NUM = float(np.log(np.float32(NB) + np.float32(1e-9)))
_LOG_DEN = float(np.log(np.float32(NB)))


def _dot(a, b):
    return jnp.dot(a, b, preferred_element_type=jnp.float32,
                   precision=lax.Precision.HIGHEST)


def _tc_body(xs_ref, xc_ref, m1_ref, o18_ref, r_ref, p_ref, out_ref):
    eps = jnp.float32(1e-9)
    s2 = _dot(xs_ref[...], m1_ref[...])
    c2 = _dot(xc_ref[...], m1_ref[...])
    cf = _dot(r_ref[...], c2)
    means = s2 / (cf + eps)
    rs = _dot(means, o18_ref[...])
    probs = means / (rs + eps)
    plogp = probs * jnp.log(probs + eps)
    e = _dot(plogp, o18_ref[...])
    mi = (jnp.float32(_LOG_NUM) + e) / jnp.float32(_LOG_DEN)
    out_ref[...] = _dot(p_ref[...], mi)


def kernel(pha, amp):
    pha = pha.astype(jnp.float32)
    amp = amp.astype(jnp.float32)
    phat = pha.reshape(2 * 8 * FP, 2, T)
    ampt = amp.reshape(2 * 8 * FA, 2, T)
    cut = jnp.linspace(-np.pi, np.pi, NB + 1).astype(jnp.float32)
    cutp = jnp.concatenate([cut, jnp.full((32 - (NB + 1),), 1e30, jnp.float32)])

    sums, cnts = _sc_hist()(phat, ampt, cutp)

    xs = sums.reshape(NW * FP * FA, BLK)
    xc = cnts.reshape(NW * FP, BLK)

    mi = pl.pallas_call(
        _tc_body,
        out_shape=jax.ShapeDtypeStruct((NW * FP * FA // 2, 1), jnp.float32),
    )(xs, xc, jnp.asarray(_M1), jnp.asarray(_ONES18), jnp.asarray(_R),
      jnp.asarray(_P))

    return mi.reshape(2, 8, FP, FA)


# speedup vs baseline: 108.6800x; 1.4035x over previous
import functools

import numpy as np
import jax
import jax.numpy as jnp
from jax import lax
from jax.experimental import pallas as pl
from jax.experimental.pallas import tpu as pltpu
from jax.experimental.pallas import tpu_sc as plsc

NB = 18
NBP = 32
FP = 8
FA = 8
T = 512
NW = 32
L = 16
CHUNKS = T // L
SUMS_W = FP * FA * NBP
CNTS_W = FP * NBP


def _sc_hist_body(pha_hbm, amp_hbm, cut_hbm, sums_hbm, cnts_hbm,
                  pha_v, amp_v, cut_v, sums_v, cnts_v, ocnt_v, sem):
    wid = lax.axis_index("s") * 2 + lax.axis_index("c")
    bc = wid % 16
    seg = wid // 16

    copies = [pltpu.async_copy(cut_hbm, cut_v, sem)]
    for fp in range(FP):
        copies.append(pltpu.async_copy(
            pha_hbm.at[bc * FP + fp, seg], pha_v.at[pl.ds(fp * T, T)], sem))
    for fa in range(FA):
        copies.append(pltpu.async_copy(
            amp_hbm.at[bc * FA + fa, seg], amp_v.at[pl.ds(fa * T, T)], sem))

    zf = jnp.zeros((L,), jnp.float32)

    def zero_sums(i, c):
        base = i * (16 * L)
        for q in range(16):
            sums_v[pl.ds(base + q * L, L)] = zf
        return c

    lax.fori_loop(0, SUMS_W // (16 * L), zero_sums, 0)
    for q in range(CNTS_W // L):
        cnts_v[pl.ds(q * L, L)] = zf

    for cp in copies:
        cp.wait()

    ones_f = jnp.ones((L,), jnp.float32)
    ones_i = jnp.ones((L,), jnp.int32)
    zero_i = jnp.zeros((L,), jnp.int32)
    pi = jnp.float32(np.pi)
    inv = jnp.float32(NB / (2.0 * np.pi))

    def chunk(ci, c):
        t0 = ci * L
        avecs = [amp_v[pl.ds(fa * T + t0, L)] for fa in range(FA)]
        for fp in range(FP):
            p = pha_v[pl.ds(fp * T + t0, L)]
            y = (p + pi) * inv + jnp.float32(32.5)
            y = jnp.minimum(jnp.maximum(y, jnp.float32(31.0)), jnp.float32(50.5))
            j = jnp.minimum(jnp.maximum(y.astype(jnp.int32) - 32, 0), NB)
            cj = plsc.load_gather(cut_v, [j])
            u = j + jnp.where(cj < p, ones_i, zero_i)
            b = jnp.minimum(jnp.maximum(u - 1, 0), NB - 1)
            plsc.addupdate_scatter(cnts_v, [fp * NBP + b], ones_f)
            fbase = fp * (FA * NBP) + b
            for fa in range(FA):
                plsc.addupdate_scatter(sums_v, [fbase + fa * NBP], avecs[fa])
        return c

    lax.fori_loop(0, CHUNKS, chunk, 0)

    for fp in range(FP):
        c0 = cnts_v[pl.ds(fp * NBP, L)]
        c1 = cnts_v[pl.ds(fp * NBP + L, L)]
        for fa in range(FA):
            ocnt_v[pl.ds((fp * FA + fa) * NBP, L)] = c0
            ocnt_v[pl.ds((fp * FA + fa) * NBP + L, L)] = c1

    pltpu.sync_copy(sums_v, sums_hbm.at[wid])
    pltpu.sync_copy(ocnt_v, cnts_hbm.at[wid])


@functools.cache
def _sc_hist():
    mesh = plsc.VectorSubcoreMesh(core_axis_name="c", subcore_axis_name="s")
    return pl.kernel(
        _sc_hist_body,
        out_type=(jax.ShapeDtypeStruct((NW, SUMS_W), jnp.float32),
                  jax.ShapeDtypeStruct((NW, SUMS_W), jnp.float32)),
        mesh=mesh,
        compiler_params=pltpu.CompilerParams(needs_layout_passes=False),
        scratch_types=(pltpu.VMEM((FP * T,), jnp.float32),
                       pltpu.VMEM((FA * T,), jnp.float32),
                       pltpu.VMEM((32,), jnp.float32),
                       pltpu.VMEM((SUMS_W,), jnp.float32),
                       pltpu.VMEM((CNTS_W,), jnp.float32),
                       pltpu.VMEM((SUMS_W,), jnp.float32),
                       pltpu.SemaphoreType.DMA),
    )


_LOG_NUM = float(np.log(np.float32(NB) + np.float32(1e-9)))
_LOG_DEN = float(np.log(np.float32(NB)))
_ONES32 = np.ones((NBP, 1), np.float32)


def _dot(a, b):
    return jnp.dot(a, b, preferred_element_type=jnp.float32,
                   precision=lax.Precision.HIGHEST)


def _tc_body(xs_ref, xc_ref, o32_ref, out_ref):
    eps = jnp.float32(1e-9)
    means = xs_ref[...] / (xc_ref[...] + eps)
    rs = _dot(means, o32_ref[...])
    probs = means / (rs + eps)
    plogp = probs * jnp.log(probs + eps)
    e = _dot(plogp, o32_ref[...])
    mi = (jnp.float32(_LOG_NUM) + e) / jnp.float32(_LOG_DEN)
    half = NW * FP * FA // 2
    out_ref[...] = (mi[0:half] + mi[half:2 * half]) * jnp.float32(0.5)


def kernel(pha, amp):
    pha = pha.astype(jnp.float32)
    amp = amp.astype(jnp.float32)
    phat = pha.reshape(2 * 8 * FP, 2, T)
    ampt = amp.reshape(2 * 8 * FA, 2, T)
    cut = jnp.linspace(-np.pi, np.pi, NB + 1).astype(jnp.float32)
    cutp = jnp.concatenate([cut, jnp.full((32 - (NB + 1),), 1e30, jnp.float32)])

    sums, cnts = _sc_hist()(phat, ampt, cutp)

    xs = sums.reshape(NW * FP * FA, NBP)
    xc = cnts.reshape(NW * FP * FA, NBP)

    mi = pl.pallas_call(
        _tc_body,
        out_shape=jax.ShapeDtypeStruct((NW * FP * FA // 2, 1), jnp.float32),
    )(xs, xc, jnp.asarray(_ONES32))

    return mi.reshape(2, 8, FP, FA)


# speedup vs baseline: 113.4390x; 1.0438x over previous
import functools

import numpy as np
import jax
import jax.numpy as jnp
from jax import lax
from jax.experimental import pallas as pl
from jax.experimental.pallas import tpu as pltpu
from jax.experimental.pallas import tpu_sc as plsc

NB = 18
NBP = 32
FP = 8
FA = 8
T = 512
NW = 32
L = 16
CHUNKS = T // L
SUMS_W = FP * FA * NBP
CNTS_W = FP * NBP

_C5 = (0.04342837, -0.4048623, 1.5938846, -3.492466, 5.046853, -2.7868056)
_LN2 = 0.6931471805599453
_EYE = np.eye(L, dtype=np.float32)


def _vlog(x):
    bits = plsc.bitcast(x, jnp.int32)
    e = lax.shift_right_arithmetic(bits, 23) - 127
    mbits = lax.bitwise_or(lax.bitwise_and(bits, 0x007FFFFF), 0x3F800000)
    m = plsc.bitcast(mbits, jnp.float32)
    acc = jnp.full((L,), _C5[0], jnp.float32)
    for c in _C5[1:]:
        acc = acc * m + jnp.float32(c)
    return (acc + e.astype(jnp.float32)) * jnp.float32(_LN2)


def _sc_mi_body(pha_hbm, amp_hbm, cut_hbm, out_hbm,
                pha_v, amp_v, cut_v, sums_v, cnts_v, mi_v, sem):
    core = lax.axis_index("c")
    sub = lax.axis_index("s")
    wid = sub * 2 + core
    bc = wid % 16
    seg = wid // 16

    copies = [pltpu.async_copy(cut_hbm, cut_v, sem)]
    for fp in range(FP):
        copies.append(pltpu.async_copy(
            pha_hbm.at[bc * FP + fp, seg], pha_v.at[pl.ds(fp * T, T)], sem))
    for fa in range(FA):
        copies.append(pltpu.async_copy(
            amp_hbm.at[bc * FA + fa, seg], amp_v.at[pl.ds(fa * T, T)], sem))

    zf = jnp.zeros((L,), jnp.float32)

    def zero_sums(i, c):
        base = i * (8 * L)
        for q in range(8):
            sums_v[pl.ds(base + q * L, L)] = zf
        return c

    lax.fori_loop(0, SUMS_W // (8 * L), zero_sums, 0)
    for q in range(CNTS_W // L):
        cnts_v[pl.ds(q * L, L)] = zf

    for cp in copies:
        cp.wait()

    ones_f = jnp.ones((L,), jnp.float32)
    ones_i = jnp.ones((L,), jnp.int32)
    zero_i = jnp.zeros((L,), jnp.int32)
    pi = jnp.float32(np.pi)
    inv = jnp.float32(NB / (2.0 * np.pi))

    def chunk(ci, c):
        t0 = ci * L
        avecs = [amp_v[pl.ds(fa * T + t0, L)] for fa in range(FA)]
        for fp in range(FP):
            p = pha_v[pl.ds(fp * T + t0, L)]
            y = (p + pi) * inv + jnp.float32(32.5)
            y = jnp.minimum(jnp.maximum(y, jnp.float32(31.0)), jnp.float32(50.5))
            j = jnp.minimum(jnp.maximum(y.astype(jnp.int32) - 32, 0), NB)
            cj = plsc.load_gather(cut_v, [j])
            u = j + jnp.where(cj < p, ones_i, zero_i)
            b = jnp.minimum(jnp.maximum(u - 1, 0), NB - 1)
            plsc.addupdate_scatter(cnts_v, [fp * NBP + b], ones_f)
            fbase = fp * (FA * NBP) + b
            for fa in range(FA):
                plsc.addupdate_scatter(sums_v, [fbase + fa * NBP], avecs[fa])
        return c

    lax.fori_loop(0, CHUNKS, chunk, 0)

    eps = jnp.float32(1e-9)
    lane = lax.iota(jnp.int32, L)
    hots = [jnp.where(lane == k, jnp.float32(1.0), jnp.float32(0.0))
            for k in range(L)]
    accs = [zf] * (FP * FA // L)
    for fp in range(FP):
        ce0 = cnts_v[pl.ds(fp * NBP, L)] + eps
        ce1 = cnts_v[pl.ds(fp * NBP + L, L)] + eps
        for fa in range(FA):
            base = (fp * FA + fa) * NBP
            m0 = sums_v[pl.ds(base, L)] / ce0
            m1 = sums_v[pl.ds(base + L, L)] / ce1
            rs = jnp.sum(m0 + m1) + eps
            p0 = m0 / rs
            p1 = m1 / rs
            ent = jnp.sum(p0 * _vlog(p0 + eps) + p1 * _vlog(p1 + eps))
            i = fp * FA + fa
            accs[i // L] = accs[i // L] + ent * hots[i % L]
    for q in range(FP * FA // L):
        mi_v[pl.ds(q * L, L)] = accs[q]

    pltpu.sync_copy(mi_v, out_hbm.at[wid])


@functools.cache
def _sc_mi():
    mesh = plsc.VectorSubcoreMesh(core_axis_name="c", subcore_axis_name="s")
    return pl.kernel(
        _sc_mi_body,
        out_type=jax.ShapeDtypeStruct((NW, FP * FA), jnp.float32),
        mesh=mesh,
        compiler_params=pltpu.CompilerParams(needs_layout_passes=False),
        scratch_types=(pltpu.VMEM((FP * T,), jnp.float32),
                       pltpu.VMEM((FA * T,), jnp.float32),
                       pltpu.VMEM((32,), jnp.float32),
                       pltpu.VMEM((SUMS_W,), jnp.float32),
                       pltpu.VMEM((CNTS_W,), jnp.float32),
                       pltpu.VMEM((FP * FA,), jnp.float32),
                       pltpu.SemaphoreType.DMA),
    )


_LOG_NUM = float(np.log(np.float32(NB) + np.float32(1e-9)))
_LOG_DEN = float(np.log(np.float32(NB)))


def _tc_body(e_ref, out_ref):
    e = (e_ref[0:16] + e_ref[16:NW]) * jnp.float32(0.5)
    out_ref[...] = (jnp.float32(_LOG_NUM) + e) / jnp.float32(_LOG_DEN)


def kernel(pha, amp):
    pha = pha.astype(jnp.float32)
    amp = amp.astype(jnp.float32)
    phat = pha.reshape(2 * 8 * FP, 2, T)
    ampt = amp.reshape(2 * 8 * FA, 2, T)
    cut = jnp.linspace(-np.pi, np.pi, NB + 1).astype(jnp.float32)
    cutp = jnp.concatenate([cut, jnp.full((32 - (NB + 1),), 1e30, jnp.float32)])

    ent = _sc_mi()(phat, ampt, cutp)

    mi = pl.pallas_call(
        _tc_body,
        out_shape=jax.ShapeDtypeStruct((16, FP * FA), jnp.float32),
    )(ent)

    return mi.reshape(2, 8, FP, FA)


# speedup vs baseline: 126.6093x; 1.1161x over previous
import functools

import numpy as np
import jax
import jax.numpy as jnp
from jax import lax
from jax.experimental import pallas as pl
from jax.experimental.pallas import tpu as pltpu
from jax.experimental.pallas import tpu_sc as plsc

NB = 18
NBP = 32
FP = 8
FA = 8
T = 512
NW = 32
L = 16
CHUNKS = T // L
SUMS_W = FP * FA * NBP
CNTS_W = FP * NBP

_C5 = (0.04342837, -0.4048623, 1.5938846, -3.492466, 5.046853, -2.7868056)
_LN2 = 0.6931471805599453
_EYE = np.eye(L, dtype=np.float32)


def _vlog(x):
    bits = plsc.bitcast(x, jnp.int32)
    e = lax.shift_right_arithmetic(bits, 23) - 127
    mbits = lax.bitwise_or(lax.bitwise_and(bits, 0x007FFFFF), 0x3F800000)
    m = plsc.bitcast(mbits, jnp.float32)
    acc = jnp.full((L,), _C5[0], jnp.float32)
    for c in _C5[1:]:
        acc = acc * m + jnp.float32(c)
    return (acc + e.astype(jnp.float32)) * jnp.float32(_LN2)


def _sc_mi_body(pha_hbm, amp_hbm, cut_hbm, out_hbm,
                pha_v, amp_v, cut_v, sums_v, cnts_v, means_v, mi_v, sem):
    core = lax.axis_index("c")
    sub = lax.axis_index("s")
    wid = sub * 2 + core
    bc = wid % 16
    seg = wid // 16

    copies = [pltpu.async_copy(cut_hbm, cut_v, sem)]
    for fp in range(FP):
        copies.append(pltpu.async_copy(
            pha_hbm.at[bc * FP + fp, seg], pha_v.at[pl.ds(fp * T, T)], sem))
    for fa in range(FA):
        copies.append(pltpu.async_copy(
            amp_hbm.at[bc * FA + fa, seg], amp_v.at[pl.ds(fa * T, T)], sem))

    zf = jnp.zeros((L,), jnp.float32)

    def zero_sums(i, c):
        base = i * (8 * L)
        for q in range(8):
            sums_v[pl.ds(base + q * L, L)] = zf
        return c

    lax.fori_loop(0, SUMS_W // (8 * L), zero_sums, 0)
    for q in range(CNTS_W // L):
        cnts_v[pl.ds(q * L, L)] = zf

    for cp in copies:
        cp.wait()

    ones_f = jnp.ones((L,), jnp.float32)
    ones_i = jnp.ones((L,), jnp.int32)
    zero_i = jnp.zeros((L,), jnp.int32)
    pi = jnp.float32(np.pi)
    inv = jnp.float32(NB / (2.0 * np.pi))

    def one_chunk(t0):
        avecs = [amp_v[pl.ds(fa * T + t0, L)] for fa in range(FA)]
        for fp in range(FP):
            p = pha_v[pl.ds(fp * T + t0, L)]
            y = (p + pi) * inv + jnp.float32(32.5)
            y = jnp.minimum(jnp.maximum(y, jnp.float32(31.0)), jnp.float32(50.5))
            j = jnp.minimum(jnp.maximum(y.astype(jnp.int32) - 32, 0), NB)
            cj = plsc.load_gather(cut_v, [j])
            u = j + jnp.where(cj < p, ones_i, zero_i)
            b = jnp.minimum(jnp.maximum(u - 1, 0), NB - 1)
            plsc.addupdate_scatter(cnts_v, [fp * NBP + b], ones_f)
            fbase = fp * (FA * NBP) + b
            for fa in range(FA):
                plsc.addupdate_scatter(sums_v, [fbase + fa * NBP], avecs[fa])

    def chunk(ci, c):
        t0 = ci * (2 * L)
        one_chunk(t0)
        one_chunk(t0 + L)
        return c

    lax.fori_loop(0, CHUNKS // 2, chunk, 0)

    eps = jnp.float32(1e-9)
    for fp in range(FP):
        ce0 = cnts_v[pl.ds(fp * NBP, L)] + eps
        ce1 = cnts_v[pl.ds(fp * NBP + L, L)] + eps
        for fa in range(FA):
            base = (fp * FA + fa) * NBP
            means_v[pl.ds(base, L)] = sums_v[pl.ds(base, L)] / ce0
            means_v[pl.ds(base + L, L)] = sums_v[pl.ds(base + L, L)] / ce1
    lane32 = lax.iota(jnp.int32, L) * NBP
    for g in range(FP * FA // L):
        gb = g * (L * NBP)
        rs = zf
        for bn in range(NB):
            rs = rs + plsc.load_gather(means_v, [lane32 + (gb + bn)])
        rse = rs + eps
        acc = zf
        for bn in range(NB):
            m = plsc.load_gather(means_v, [lane32 + (gb + bn)])
            pr = m / rse
            acc = acc + pr * _vlog(pr + eps)
        mi_v[pl.ds(g * L, L)] = acc

    pltpu.sync_copy(mi_v, out_hbm.at[wid])


@functools.cache
def _sc_mi():
    mesh = plsc.VectorSubcoreMesh(core_axis_name="c", subcore_axis_name="s")
    return pl.kernel(
        _sc_mi_body,
        out_type=jax.ShapeDtypeStruct((NW, FP * FA), jnp.float32),
        mesh=mesh,
        compiler_params=pltpu.CompilerParams(needs_layout_passes=False),
        scratch_types=(pltpu.VMEM((FP * T,), jnp.float32),
                       pltpu.VMEM((FA * T,), jnp.float32),
                       pltpu.VMEM((32,), jnp.float32),
                       pltpu.VMEM((SUMS_W,), jnp.float32),
                       pltpu.VMEM((CNTS_W,), jnp.float32),
                       pltpu.VMEM((SUMS_W,), jnp.float32),
                       pltpu.VMEM((FP * FA,), jnp.float32),
                       pltpu.SemaphoreType.DMA),
    )


_LOG_NUM = float(np.log(np.float32(NB) + np.float32(1e-9)))
_LOG_DEN = float(np.log(np.float32(NB)))


def _tc_body(e_ref, out_ref):
    e = (e_ref[0:16] + e_ref[16:NW]) * jnp.float32(0.5)
    out_ref[...] = (jnp.float32(_LOG_NUM) + e) / jnp.float32(_LOG_DEN)


def kernel(pha, amp):
    pha = pha.astype(jnp.float32)
    amp = amp.astype(jnp.float32)
    phat = pha.reshape(2 * 8 * FP, 2, T)
    ampt = amp.reshape(2 * 8 * FA, 2, T)
    cut = jnp.linspace(-np.pi, np.pi, NB + 1).astype(jnp.float32)
    cutp = jnp.concatenate([cut, jnp.full((32 - (NB + 1),), 1e30, jnp.float32)])

    ent = _sc_mi()(phat, ampt, cutp)

    mi = pl.pallas_call(
        _tc_body,
        out_shape=jax.ShapeDtypeStruct((16, FP * FA), jnp.float32),
    )(ent)

    return mi.reshape(2, 8, FP, FA)
